# baseline (device time: 42348 ns/iter reference)
import jax
import jax.numpy as jnp
from jax import lax
from jax.experimental import pallas as pl
from jax.experimental.pallas import tpu as pltpu

N_DEV = 4
B, S, H, Dh, Dr = 2, 256, 16, 64, 32
D = 1024
DC = 64
BS = B * S
DHR = Dh + Dr
bf16 = jnp.bfloat16
f32 = jnp.float32


def kernel(x, Wdkv, Wuk, Wuv, Wq, Wqr, Wkr, Wo):
    def body(x_ref, wdkv_ref, wuk_ref, wuv_ref, wq_ref, wqr_ref, wkr_ref,
             wo_ref, out_ref, cbuf, ukbuf, uvbuf, wqc_scr, kc_scr, o_scr,
             send_sems, recv_sems):
        me = lax.axis_index("i")

        barrier = pltpu.get_barrier_semaphore()
        for k in range(1, N_DEV):
            pl.semaphore_signal(
                barrier, inc=1,
                device_id=((me + k) % N_DEV,),
                device_id_type=pl.DeviceIdType.MESH,
            )
        pl.semaphore_wait(barrier, N_DEV - 1)

        def send(buf, t):
            rdmas = []
            for k in range(1, N_DEV):
                rdma = pltpu.make_async_remote_copy(
                    src_ref=buf.at[me],
                    dst_ref=buf.at[me],
                    send_sem=send_sems.at[k - 1, t],
                    recv_sem=recv_sems.at[t, k - 1],
                    device_id=((me + k) % N_DEV,),
                    device_id_type=pl.DeviceIdType.MESH,
                )
                rdma.start()
                rdmas.append(rdma)
            return rdmas

        ukbuf[pl.ds(me, 1)] = wuk_ref[...].astype(bf16)[None]
        uvbuf[pl.ds(me, 1)] = wuv_ref[...].astype(bf16)[None]
        sends = send(ukbuf, 1) + send(uvbuf, 2)

        x2 = x_ref[...].reshape(BS, D).astype(bf16)
        c = jnp.dot(x2, wdkv_ref[...].astype(bf16),
                    preferred_element_type=f32).astype(bf16)
        cbuf[pl.ds(me, 1)] = c[None]
        sends += send(cbuf, 0)

        scale = jnp.asarray((Dh + Dr) ** -0.5, bf16)
        wq16 = wq_ref[...].astype(bf16) * scale
        wqr16 = wqr_ref[...].astype(bf16) * scale
        for h in range(H):
            p0 = h * DHR
            wqc_scr[:, p0:p0 + Dh] = wq16[:, h * Dh:(h + 1) * Dh]
            wqc_scr[:, p0 + Dh:p0 + DHR] = wqr16[:, h * Dr:(h + 1) * Dr]
        QC = jnp.dot(x2, wqc_scr[...],
                     preferred_element_type=f32).astype(bf16)
        Kr = jnp.dot(x2, wkr_ref[...].astype(bf16),
                     preferred_element_type=f32).astype(bf16)
        for h in range(H):
            p0 = h * DHR
            kc_scr[:, p0 + Dh:p0 + DHR] = Kr
        wo16 = wo_ref[...].astype(bf16)

        for k in range(1, N_DEV):
            origin = (me - k) % N_DEV
            for t, buf in enumerate((cbuf, ukbuf, uvbuf)):
                pltpu.make_async_remote_copy(
                    src_ref=buf.at[origin],
                    dst_ref=buf.at[origin],
                    send_sem=send_sems.at[k - 1, t],
                    recv_sem=recv_sems.at[t, k - 1],
                    device_id=(me,),
                    device_id_type=pl.DeviceIdType.MESH,
                ).wait_recv()

        K = jnp.dot(cbuf[0], ukbuf[0], preferred_element_type=f32)
        V = jnp.dot(cbuf[0], uvbuf[0], preferred_element_type=f32)
        for j in range(1, N_DEV):
            cj = cbuf[j]
            K = K + jnp.dot(cj, ukbuf[j], preferred_element_type=f32)
            V = V + jnp.dot(cj, uvbuf[j], preferred_element_type=f32)
        K = K.astype(bf16)
        V16 = V.astype(bf16)
        for h in range(H):
            p0 = h * DHR
            kc_scr[:, p0:p0 + Dh] = K[:, h * Dh:(h + 1) * Dh]

        nt = (((1,), (1,)), ((), ()))
        for b in range(B):
            r0 = b * S
            for h in range(H):
                p0 = h * DHR
                qc = QC[r0:r0 + S, p0:p0 + DHR]
                kc = kc_scr[r0:r0 + S, p0:p0 + DHR]
                s = lax.dot_general(qc, kc, nt,
                                    preferred_element_type=f32).astype(bf16)
                m = jnp.max(s, axis=1, keepdims=True)
                e = jnp.exp(s - m)
                r = jnp.sum(e, axis=1, keepdims=True, dtype=f32)
                p = e * (1.0 / r).astype(bf16)
                o = jnp.dot(p, V16[r0:r0 + S, h * Dh:(h + 1) * Dh],
                            preferred_element_type=f32)
                o_scr[r0:r0 + S, h * Dh:(h + 1) * Dh] = o.astype(bf16)

        out = jnp.dot(o_scr[...], wo16, preferred_element_type=f32)
        out_ref[...] = out.reshape(B, S, D)

        for rdma in sends:
            rdma.wait_send()

    return pl.pallas_call(
        body,
        out_shape=jax.ShapeDtypeStruct((B, S, D), f32),
        in_specs=[pl.BlockSpec(memory_space=pltpu.VMEM)] * 8,
        out_specs=pl.BlockSpec(memory_space=pltpu.VMEM),
        scratch_shapes=[
            pltpu.VMEM((N_DEV, BS, DC), bf16),
            pltpu.VMEM((N_DEV, DC, D), bf16),
            pltpu.VMEM((N_DEV, DC, D), bf16),
            pltpu.VMEM((D, H * DHR), bf16),
            pltpu.VMEM((BS, H * DHR), bf16),
            pltpu.VMEM((BS, H * Dh), bf16),
            pltpu.SemaphoreType.DMA((N_DEV - 1, 3)),
            pltpu.SemaphoreType.DMA((3, N_DEV - 1)),
        ],
        compiler_params=pltpu.CompilerParams(collective_id=0),
    )(x, Wdkv, Wuk, Wuv, Wq, Wqr, Wkr, Wo)


# device time: 28809 ns/iter; 1.4700x vs baseline; 1.4700x over previous
import jax
import jax.numpy as jnp
from jax import lax
from jax.experimental import pallas as pl
from jax.experimental.pallas import tpu as pltpu

N_DEV = 4
B, S, H, Dh, Dr = 2, 256, 16, 64, 32
D = 1024
DC = 64
BS = B * S
DHR = Dh + Dr
bf16 = jnp.bfloat16
f32 = jnp.float32


def kernel(x, Wdkv, Wuk, Wuv, Wq, Wqr, Wkr, Wo):
    def body(x_ref, wdkv_ref, wuk_ref, wuv_ref, wq_ref, wqr_ref, wkr_ref,
             wo_ref, out_ref, cbuf, ukbuf, uvbuf, wqc_scr, kc_scr,
             vp_scr, o_scr, xs, wdkvs, wuks, wuvs, wqs, wqrs, wkrs, wos,
             outs, load_sems, send_sems, recv_sems):
        me = lax.axis_index("i")

        loads = []
        for i, (hbm, vmem) in enumerate([
                (wuk_ref, wuks), (wuv_ref, wuvs), (x_ref, xs),
                (wdkv_ref, wdkvs), (wq_ref, wqs), (wqr_ref, wqrs),
                (wkr_ref, wkrs), (wo_ref, wos)]):
            dma = pltpu.make_async_copy(hbm, vmem, load_sems.at[i])
            dma.start()
            loads.append(dma)
        (ld_uk, ld_uv, ld_x, ld_dkv, ld_q, ld_qr, ld_kr, ld_o) = loads

        barrier = pltpu.get_barrier_semaphore()
        for k in range(1, N_DEV):
            pl.semaphore_signal(
                barrier, inc=1,
                device_id=((me + k) % N_DEV,),
                device_id_type=pl.DeviceIdType.MESH,
            )

        def send(buf, t):
            rdmas = []
            for k in range(1, N_DEV):
                rdma = pltpu.make_async_remote_copy(
                    src_ref=buf.at[0],
                    dst_ref=buf.at[k],
                    send_sem=send_sems.at[k - 1, t],
                    recv_sem=recv_sems.at[t, k - 1],
                    device_id=((me + k) % N_DEV,),
                    device_id_type=pl.DeviceIdType.MESH,
                )
                rdma.start()
                rdmas.append(rdma)
            return rdmas

        ld_uk.wait()
        uk16 = wuks[...].astype(bf16)
        ukbuf[0] = uk16
        ld_uv.wait()
        uv16 = wuvs[...].astype(bf16)
        uvbuf[0] = uv16
        pl.semaphore_wait(barrier, N_DEV - 1)
        sends = send(ukbuf, 1) + send(uvbuf, 2)

        ld_x.wait()
        x2 = xs[...].reshape(BS, D).astype(bf16)
        ld_dkv.wait()
        wdkv2 = wdkvs[...]
        c = (jnp.dot(x2[:, :512], wdkv2[:, :DC], preferred_element_type=f32)
             + jnp.dot(x2[:, 512:], wdkv2[:, DC:],
                       preferred_element_type=f32)).astype(bf16)
        cbuf[0] = c
        sends += send(cbuf, 0)

        scale = jnp.asarray((Dh + Dr) ** -0.5 * 1.4426950408889634, bf16)
        ld_q.wait()
        wq16 = wqs[...].astype(bf16) * scale
        ld_qr.wait()
        wqr16 = wqrs[...].astype(bf16) * scale
        for h in range(H):
            p0 = h * DHR
            wqc_scr[:, p0:p0 + Dh] = wq16[:, h * Dh:(h + 1) * Dh]
            wqc_scr[:, p0 + Dh:p0 + DHR] = wqr16[:, h * Dr:(h + 1) * Dr]
        QC = jnp.dot(x2, wqc_scr[...],
                     preferred_element_type=f32).astype(bf16)
        ld_kr.wait()
        wkr2 = wkrs[...]
        Kr_acc = jnp.dot(x2[:, :256], wkr2[:, :Dr],
                         preferred_element_type=f32)
        for i in range(1, 4):
            Kr_acc = Kr_acc + jnp.dot(x2[:, i * 256:(i + 1) * 256],
                                      wkr2[:, i * Dr:(i + 1) * Dr],
                                      preferred_element_type=f32)
        Kr = Kr_acc.astype(bf16)
        for h in range(H):
            p0 = h * DHR
            kc_scr[:, p0 + Dh:p0 + DHR] = Kr
        K0 = jnp.dot(c, uk16, preferred_element_type=f32)
        V0 = jnp.dot(c, uv16, preferred_element_type=f32)
        ld_o.wait()
        wo16 = wos[...].astype(bf16)
        ones_col = jnp.ones((BS, 1), bf16)
        for h in range(H):
            vp_scr[:, h * 128 + Dh:h * 128 + Dh + 1] = ones_col

        for k in range(1, N_DEV):
            for t, buf in enumerate((cbuf, ukbuf, uvbuf)):
                pltpu.make_async_remote_copy(
                    src_ref=buf.at[0],
                    dst_ref=buf.at[k],
                    send_sem=send_sems.at[k - 1, t],
                    recv_sem=recv_sems.at[t, k - 1],
                    device_id=(me,),
                    device_id_type=pl.DeviceIdType.MESH,
                ).wait_recv()

        K = K0
        V = V0
        for j in range(1, N_DEV):
            cj = cbuf[j]
            K = K + jnp.dot(cj, ukbuf[j], preferred_element_type=f32)
            V = V + jnp.dot(cj, uvbuf[j], preferred_element_type=f32)
        K = K.astype(bf16)
        V16 = V.astype(bf16)
        for h in range(H):
            p0 = h * DHR
            kc_scr[:, p0:p0 + Dh] = K[:, h * Dh:(h + 1) * Dh]
            vp_scr[:, h * 128:h * 128 + Dh] = V16[:, h * Dh:(h + 1) * Dh]

        nt = (((1,), (1,)), ((), ()))
        out_stores = []
        for b in range(B):
            r0 = b * S
            for h in range(H):
                p0 = h * DHR
                qc = QC[r0:r0 + S, p0:p0 + DHR]
                kc = kc_scr[r0:r0 + S, p0:p0 + DHR]
                s = lax.dot_general(qc, kc, nt,
                                    preferred_element_type=f32).astype(bf16)
                e = jnp.exp2(s)
                o2 = jnp.dot(e, vp_scr[r0:r0 + S, h * 128:(h + 1) * 128],
                             preferred_element_type=f32)
                o = o2[:, :Dh] * (1.0 / o2[:, Dh:Dh + 1])
                o_scr[r0:r0 + S, h * Dh:(h + 1) * Dh] = o.astype(bf16)

            out_b = jnp.dot(o_scr[r0:r0 + S, :], wo16,
                            preferred_element_type=f32)
            outs[b] = out_b.astype(bf16)
            store = pltpu.make_async_copy(outs.at[b], out_ref.at[b],
                                          load_sems.at[8 + b])
            store.start()
            out_stores.append(store)

        for rdma in sends:
            rdma.wait_send()
        for store in out_stores:
            store.wait()

    Wdkv = jnp.concatenate([Wdkv[:512], Wdkv[512:]], axis=1).astype(bf16)
    Wkr = jnp.concatenate([Wkr[i * 256:(i + 1) * 256] for i in range(4)],
                          axis=1).astype(bf16)
    out16 = pl.pallas_call(
        body,
        out_shape=jax.ShapeDtypeStruct((B, S, D), bf16),
        in_specs=[pl.BlockSpec(memory_space=pl.ANY)] * 8,
        out_specs=pl.BlockSpec(memory_space=pl.ANY),
        scratch_shapes=[
            pltpu.VMEM((N_DEV, BS, DC), bf16),
            pltpu.VMEM((N_DEV, DC, D), bf16),
            pltpu.VMEM((N_DEV, DC, D), bf16),
            pltpu.VMEM((D, H * DHR), bf16),
            pltpu.VMEM((BS, H * DHR), bf16),
            pltpu.VMEM((BS, H * 128), bf16),
            pltpu.VMEM((BS, H * Dh), bf16),
            pltpu.VMEM((B, S, D), f32),
            pltpu.VMEM((512, 128), bf16),
            pltpu.VMEM((DC, D), f32),
            pltpu.VMEM((DC, D), f32),
            pltpu.VMEM((D, D), f32),
            pltpu.VMEM((D, H * Dr), f32),
            pltpu.VMEM((256, 128), bf16),
            pltpu.VMEM((D, D), f32),
            pltpu.VMEM((B, S, D), bf16),
            pltpu.SemaphoreType.DMA((10,)),
            pltpu.SemaphoreType.DMA((N_DEV - 1, 3)),
            pltpu.SemaphoreType.DMA((3, N_DEV - 1)),
        ],
        compiler_params=pltpu.CompilerParams(
            collective_id=0, vmem_limit_bytes=64 * 1024 * 1024),
    )(x, Wdkv, Wuk, Wuv, Wq, Wqr, Wkr, Wo)
    return out16.astype(f32)
